# Initial kernel scaffold; baseline (speedup 1.0000x reference)
import jax
import jax.numpy as jnp
from jax.experimental import pallas as pl


def kernel(in_pc, raw_w_weights, weights, bias, neighbor_id_lstlst):
    raise NotImplementedError("write your pallas kernel here")


# speedup vs baseline: 3.3843x; 3.3843x over previous
import functools

import jax
import jax.numpy as jnp
import numpy as np
from jax import lax
from jax.experimental import pallas as pl
from jax.experimental.pallas import tpu as pltpu
from jax.experimental.pallas import tpu_sc as plsc

BATCH = 1
IN_PN = 10000
OUT_PN = 10000
MAX_NB = 32
IN_CH = 128
OUT_CH = 128
WN = 4

L = 16
NC = 2
NS = 16
NW = NC * NS
PTS_PER_W = 320
P_PAD = NW * PTS_PER_W
CHUNK_PTS = 4
EDGES_PER_CHUNK = CHUNK_PTS * MAX_NB
NCHUNK = PTS_PER_W // CHUNK_PTS
ZDIM = WN * IN_CH
PAD_ID = IN_PN

_TAKE_DNUMS = lax.GatherDimensionNumbers(
    offset_dims=(), collapsed_slice_dims=(0,), start_index_map=(0,))


def _take(vec, idx):
    return lax.gather(vec, idx[:, None], dimension_numbers=_TAKE_DNUMS,
                      slice_sizes=(1,),
                      mode=lax.GatherScatterMode.PROMISE_IN_BOUNDS)


def _bcast_idx(lane):
    return jnp.full((L,), lane, jnp.int32)


def _expand_idx(j):
    return (4 * j) % L + lax.shift_right_logical(lax.iota(jnp.int32, L), 2)


def _sc_fuse_body(x_hbm, ids_hbm, w_hbm, z_hbm, ids_v, w_v, clamped_v,
                  rows_v, zstage_v, sem):
    wid = lax.axis_index("s") * NC + lax.axis_index("c")
    base = wid * PTS_PER_W

    pltpu.sync_copy(ids_hbm.at[pl.ds(base, PTS_PER_W)], ids_v)
    pltpu.sync_copy(w_hbm.at[pl.ds(base, PTS_PER_W)], w_v)

    def mask_body(p, carry):
        idv = [ids_v[p, pl.ds(16 * k, L)] for k in range(2)]
        for j in range(IN_CH // L):
            ids_exp = _take(idv[j // 4], _expand_idx(j))
            wv = w_v[p, pl.ds(L * j, L)]
            m = jnp.minimum(jnp.abs(ids_exp - PAD_ID), 1).astype(jnp.float32)
            w_v[p, pl.ds(L * j, L)] = wv * m
        return carry

    lax.fori_loop(0, PTS_PER_W, mask_body, 0)

    def clamp_body(g, carry):
        for v in range(EDGES_PER_CHUNK // L):
            ids = ids_v[CHUNK_PTS * g + v // 2, pl.ds(16 * (v % 2), L)]
            clamped_v[g, pl.ds(L * v, L)] = jnp.minimum(ids, IN_PN - 1)
        return carry

    lax.fori_loop(0, NCHUNK, clamp_body, 0)

    def chunk_body(g, carry):
        pltpu.async_copy(x_hbm.at[clamped_v.at[g]], rows_v, sem).wait()

        def point_body(pp, c2):
            p = CHUNK_PTS * g + pp
            acc = [jnp.zeros((L,), jnp.float32) for _ in range(4 * 8)]
            for n in range(MAX_NB):
                if n % 4 == 0:
                    wv = w_v[p, pl.ds(L * (n // 4), L)]
                r = MAX_NB * pp + n
                row = [rows_v[r, pl.ds(L * j, L)] for j in range(8)]
                for m in range(WN):
                    wb = _take(wv, _bcast_idx(4 * (n % 4) + m))
                    for j in range(8):
                        acc[8 * m + j] = acc[8 * m + j] + wb * row[j]
            for m in range(WN):
                for j in range(8):
                    zstage_v[pp, pl.ds(IN_CH * m + L * j, L)] = acc[8 * m + j]
            return c2

        lax.fori_loop(0, CHUNK_PTS, point_body, 0)
        pltpu.sync_copy(zstage_v, z_hbm.at[pl.ds(base + CHUNK_PTS * g,
                                                 CHUNK_PTS)])
        return carry

    lax.fori_loop(0, NCHUNK, chunk_body, 0)


@jax.jit
def _sc_fuse(x, ids_pad, w_pad):
    mesh = plsc.VectorSubcoreMesh(core_axis_name="c", subcore_axis_name="s")
    return pl.kernel(
        _sc_fuse_body,
        out_type=jax.ShapeDtypeStruct((P_PAD, ZDIM), jnp.float32),
        mesh=mesh,
        scratch_types=[
            pltpu.VMEM((PTS_PER_W, MAX_NB), jnp.int32),
            pltpu.VMEM((PTS_PER_W, IN_CH), jnp.float32),
            pltpu.VMEM((NCHUNK, EDGES_PER_CHUNK), jnp.int32),
            pltpu.VMEM((EDGES_PER_CHUNK, IN_CH), jnp.float32),
            pltpu.VMEM((CHUNK_PTS, ZDIM), jnp.float32),
            pltpu.SemaphoreType.DMA,
        ],
    )(x, ids_pad, w_pad)


def _tc_body(z_ref, w_ref, b_ref, o_ref):
    y = jnp.dot(z_ref[...], w_ref[...],
                preferred_element_type=jnp.float32) + b_ref[...]
    o_ref[...] = jnp.where(y > 0.0, y, jnp.exp(jnp.minimum(y, 0.0)) - 1.0)


@jax.jit
def _tc_matmul(z, wbig, bias_pad):
    blk = 512
    grid = P_PAD // blk
    return pl.pallas_call(
        _tc_body,
        grid=(grid,),
        in_specs=[
            pl.BlockSpec((blk, ZDIM), lambda i: (i, 0)),
            pl.BlockSpec((ZDIM, OUT_CH), lambda i: (0, 0)),
            pl.BlockSpec((blk, OUT_CH), lambda i: (i, 0)),
        ],
        out_specs=pl.BlockSpec((blk, OUT_CH), lambda i: (i, 0)),
        out_shape=jax.ShapeDtypeStruct((P_PAD, OUT_CH), jnp.float32),
    )(z, wbig, bias_pad)


def kernel(in_pc, raw_w_weights, weights, bias, neighbor_id_lstlst):
    x = in_pc.reshape(IN_PN, IN_CH)
    ids_pad = jnp.pad(neighbor_id_lstlst, ((0, P_PAD - OUT_PN), (0, 0)),
                      constant_values=PAD_ID)
    w_pad = jnp.pad(raw_w_weights.reshape(OUT_PN, MAX_NB * WN),
                    ((0, P_PAD - OUT_PN), (0, 0)))
    wbig = weights.reshape(WN, OUT_CH, IN_CH).transpose(0, 2, 1).reshape(
        ZDIM, OUT_CH)
    bias_pad = jnp.pad(bias, ((0, P_PAD - OUT_PN), (0, 0)))

    z = _sc_fuse(x, ids_pad, w_pad)
    out = _tc_matmul(z, wbig, bias_pad)
    return out[:OUT_PN].reshape(BATCH, OUT_PN, OUT_CH)


# speedup vs baseline: 3.4159x; 1.0093x over previous
import functools

import jax
import jax.numpy as jnp
import numpy as np
from jax import lax
from jax.experimental import pallas as pl
from jax.experimental.pallas import tpu as pltpu
from jax.experimental.pallas import tpu_sc as plsc

BATCH = 1
IN_PN = 10000
OUT_PN = 10000
MAX_NB = 32
IN_CH = 128
OUT_CH = 128
WN = 4

L = 16
NC = 2
NS = 16
NW = NC * NS
PTS_PER_W = 320
P_PAD = NW * PTS_PER_W
CHUNK_PTS = 4
EDGES_PER_CHUNK = CHUNK_PTS * MAX_NB
NCHUNK = PTS_PER_W // CHUNK_PTS
ZDIM = WN * IN_CH
PAD_ID = IN_PN

_TAKE_DNUMS = lax.GatherDimensionNumbers(
    offset_dims=(), collapsed_slice_dims=(0,), start_index_map=(0,))


def _take(vec, idx):
    return lax.gather(vec, idx[:, None], dimension_numbers=_TAKE_DNUMS,
                      slice_sizes=(1,),
                      mode=lax.GatherScatterMode.PROMISE_IN_BOUNDS)


def _bcast_idx(lane):
    return jnp.full((L,), lane, jnp.int32)


def _expand_idx(j):
    return (4 * j) % L + lax.shift_right_logical(lax.iota(jnp.int32, L), 2)


def _sc_fuse_body(x_hbm, ids_hbm, w_hbm, z_hbm, w_v, clamped_v, rows_v,
                  zstage_v, sem_g0, sem_g1, sem_o):
    wid = lax.axis_index("s") * NC + lax.axis_index("c")
    base = wid * PTS_PER_W
    gbase = wid * NCHUNK

    pltpu.sync_copy(ids_hbm.at[pl.ds(gbase, NCHUNK)],
                    clamped_v.at[pl.ds(0, NCHUNK)])
    pltpu.sync_copy(w_hbm.at[pl.ds(base, PTS_PER_W)], w_v)
    for k in range(2):
        for v in range(EDGES_PER_CHUNK // L):
            clamped_v[NCHUNK + k, pl.ds(L * v, L)] = jnp.zeros((L,),
                                                               jnp.int32)

    def prep_body(g, carry):
        for pp in range(CHUNK_PTS):
            p = CHUNK_PTS * g + pp
            idv = [clamped_v[g, pl.ds(MAX_NB * pp + L * k, L)]
                   for k in range(2)]
            for j in range(IN_CH // L):
                ids_exp = _take(idv[j // 4], _expand_idx(j))
                m = jnp.minimum(jnp.abs(ids_exp - PAD_ID), 1).astype(
                    jnp.float32)
                w_v[p, pl.ds(L * j, L)] = w_v[p, pl.ds(L * j, L)] * m
        for v in range(EDGES_PER_CHUNK // L):
            ids = clamped_v[g, pl.ds(L * v, L)]
            clamped_v[g, pl.ds(L * v, L)] = jnp.minimum(ids, IN_PN - 1)
        return carry

    lax.fori_loop(0, NCHUNK, prep_body, 0)

    sems = (sem_g0, sem_g1)
    for half in range(2):
        pltpu.async_copy(x_hbm.at[clamped_v.at[half]], rows_v.at[half],
                         sems[half])

    def pair_body(t, carry):
        for half in range(2):
            g = 2 * t + half
            pltpu.make_async_copy(x_hbm.at[clamped_v.at[g]],
                                  rows_v.at[half], sems[half]).wait()

            @pl.when(t > 0)
            def _():
                pltpu.make_async_copy(
                    zstage_v.at[half],
                    z_hbm.at[pl.ds(base + CHUNK_PTS * (g - 2), CHUNK_PTS)],
                    sem_o).wait()

            def point_body(pp, c2):
                p = CHUNK_PTS * g + pp
                acc = [jnp.zeros((L,), jnp.float32) for _ in range(4 * 8)]
                for n in range(MAX_NB):
                    if n % 4 == 0:
                        wv = w_v[p, pl.ds(L * (n // 4), L)]
                    r = MAX_NB * pp + n
                    row = [rows_v[half, r, pl.ds(L * j, L)]
                           for j in range(8)]
                    for m in range(WN):
                        wb = _take(wv, _bcast_idx(4 * (n % 4) + m))
                        for j in range(8):
                            acc[8 * m + j] = acc[8 * m + j] + wb * row[j]
                for m in range(WN):
                    for j in range(8):
                        zstage_v[half, pp,
                                 pl.ds(IN_CH * m + L * j, L)] = acc[8 * m + j]
                return c2

            lax.fori_loop(0, CHUNK_PTS, point_body, 0)
            pltpu.async_copy(
                zstage_v.at[half],
                z_hbm.at[pl.ds(base + CHUNK_PTS * g, CHUNK_PTS)], sem_o)
            pltpu.async_copy(x_hbm.at[clamped_v.at[g + 2]], rows_v.at[half],
                             sems[half])
        return carry

    lax.fori_loop(0, NCHUNK // 2, pair_body, 0)

    for half in range(2):
        pltpu.make_async_copy(x_hbm.at[clamped_v.at[NCHUNK + half]],
                              rows_v.at[half], sems[half]).wait()
        pltpu.make_async_copy(
            zstage_v.at[half],
            z_hbm.at[pl.ds(base + CHUNK_PTS * (NCHUNK - 2 + half),
                           CHUNK_PTS)], sem_o).wait()


@jax.jit
def _sc_fuse(x, ids_pad, w_pad):
    mesh = plsc.VectorSubcoreMesh(core_axis_name="c", subcore_axis_name="s")
    return pl.kernel(
        _sc_fuse_body,
        out_type=jax.ShapeDtypeStruct((P_PAD, ZDIM), jnp.float32),
        mesh=mesh,
        scratch_types=[
            pltpu.VMEM((PTS_PER_W, IN_CH), jnp.float32),
            pltpu.VMEM((NCHUNK + 2, EDGES_PER_CHUNK), jnp.int32),
            pltpu.VMEM((2, EDGES_PER_CHUNK, IN_CH), jnp.float32),
            pltpu.VMEM((2, CHUNK_PTS, ZDIM), jnp.float32),
            pltpu.SemaphoreType.DMA,
            pltpu.SemaphoreType.DMA,
            pltpu.SemaphoreType.DMA,
        ],
    )(x, ids_pad, w_pad)


def _tc_body(z_ref, w_ref, b_ref, o_ref):
    y = jnp.dot(z_ref[...], w_ref[...],
                preferred_element_type=jnp.float32) + b_ref[...]
    o_ref[...] = jnp.where(y > 0.0, y, jnp.exp(jnp.minimum(y, 0.0)) - 1.0)


@jax.jit
def _tc_matmul(z, wbig, bias_pad):
    blk = 512
    grid = P_PAD // blk
    return pl.pallas_call(
        _tc_body,
        grid=(grid,),
        in_specs=[
            pl.BlockSpec((blk, ZDIM), lambda i: (i, 0)),
            pl.BlockSpec((ZDIM, OUT_CH), lambda i: (0, 0)),
            pl.BlockSpec((blk, OUT_CH), lambda i: (i, 0)),
        ],
        out_specs=pl.BlockSpec((blk, OUT_CH), lambda i: (i, 0)),
        out_shape=jax.ShapeDtypeStruct((P_PAD, OUT_CH), jnp.float32),
    )(z, wbig, bias_pad)


def kernel(in_pc, raw_w_weights, weights, bias, neighbor_id_lstlst):
    x = in_pc.reshape(IN_PN, IN_CH)
    ids_pad = jnp.pad(neighbor_id_lstlst, ((0, P_PAD - OUT_PN), (0, 0)),
                      constant_values=PAD_ID).reshape(
                          P_PAD // CHUNK_PTS, EDGES_PER_CHUNK)
    w_pad = jnp.pad(raw_w_weights.reshape(OUT_PN, MAX_NB * WN),
                    ((0, P_PAD - OUT_PN), (0, 0)))
    wbig = weights.reshape(WN, OUT_CH, IN_CH).transpose(0, 2, 1).reshape(
        ZDIM, OUT_CH)
    bias_pad = jnp.pad(bias, ((0, P_PAD - OUT_PN), (0, 0)))

    z = _sc_fuse(x, ids_pad, w_pad)
    out = _tc_matmul(z, wbig, bias_pad)
    return out[:OUT_PN].reshape(BATCH, OUT_PN, OUT_CH)


# speedup vs baseline: 3.4816x; 1.0192x over previous
import functools

import jax
import jax.numpy as jnp
import numpy as np
from jax import lax
from jax.experimental import pallas as pl
from jax.experimental.pallas import tpu as pltpu
from jax.experimental.pallas import tpu_sc as plsc

BATCH = 1
IN_PN = 10000
OUT_PN = 10000
MAX_NB = 32
IN_CH = 128
OUT_CH = 128
WN = 4

L = 16
NC = 2
NS = 16
NW = NC * NS
PTS_PER_W = 320
P_PAD = NW * PTS_PER_W
CHUNK_PTS = 4
EDGES_PER_CHUNK = CHUNK_PTS * MAX_NB
NCHUNK = PTS_PER_W // CHUNK_PTS
ZDIM = WN * IN_CH
PAD_ID = IN_PN

_TAKE_DNUMS = lax.GatherDimensionNumbers(
    offset_dims=(), collapsed_slice_dims=(0,), start_index_map=(0,))


def _take(vec, idx):
    return lax.gather(vec, idx[:, None], dimension_numbers=_TAKE_DNUMS,
                      slice_sizes=(1,),
                      mode=lax.GatherScatterMode.PROMISE_IN_BOUNDS)


def _bcast_idx(lane):
    return jnp.full((L,), lane, jnp.int32)


def _expand_idx(j):
    return (4 * j) % L + lax.shift_right_logical(lax.iota(jnp.int32, L), 2)


def _sc_fuse_body(x_hbm, ids_hbm, w_hbm, z_hbm, w_v, clamped_v, rows_v,
                  zstage_v, sem_g0, sem_g1, sem_o):
    wid = lax.axis_index("s") * NC + lax.axis_index("c")
    base = wid * PTS_PER_W
    gbase = wid * NCHUNK

    pltpu.sync_copy(ids_hbm.at[pl.ds(gbase, NCHUNK)],
                    clamped_v.at[pl.ds(0, NCHUNK)])
    pltpu.sync_copy(w_hbm.at[pl.ds(base, PTS_PER_W)], w_v)
    for k in range(2):
        for v in range(EDGES_PER_CHUNK // L):
            clamped_v[NCHUNK + k, v // 2,
                      pl.ds(L * (v % 2), L)] = jnp.zeros((L,), jnp.int32)

    def prep_body(g, carry):
        for pp in range(CHUNK_PTS):
            p = CHUNK_PTS * g + pp
            idv = [clamped_v[g, pp, pl.ds(L * k, L)] for k in range(2)]
            for j in range(IN_CH // L):
                ids_exp = _take(idv[j // 4], _expand_idx(j))
                m = jnp.minimum(jnp.abs(ids_exp - PAD_ID), 1).astype(
                    jnp.float32)
                w_v[p, pl.ds(L * j, L)] = w_v[p, pl.ds(L * j, L)] * m
        for v in range(EDGES_PER_CHUNK // L):
            ids = clamped_v[g, v // 2, pl.ds(L * (v % 2), L)]
            clamped_v[g, v // 2, pl.ds(L * (v % 2), L)] = jnp.minimum(
                ids, IN_PN - 1)
        return carry

    lax.fori_loop(0, NCHUNK, prep_body, 0)

    sems = (sem_g0, sem_g1)
    NSTREAM = 4
    ROWS_PER_STREAM = EDGES_PER_CHUNK // NSTREAM

    def _issue_gather(g, half):
        for q in range(NSTREAM):
            pltpu.async_copy(
                x_hbm.at[clamped_v.at[g, q]],
                rows_v.at[half, pl.ds(ROWS_PER_STREAM * q, ROWS_PER_STREAM)],
                sems[half])

    def _wait_gather(g, half):
        for q in range(NSTREAM):
            pltpu.make_async_copy(
                x_hbm.at[clamped_v.at[g, q]],
                rows_v.at[half, pl.ds(ROWS_PER_STREAM * q, ROWS_PER_STREAM)],
                sems[half]).wait()

    for half in range(2):
        _issue_gather(half, half)

    def pair_body(t, carry):
        for half in range(2):
            g = 2 * t + half
            _wait_gather(g, half)

            @pl.when(t > 0)
            def _():
                pltpu.make_async_copy(
                    zstage_v.at[half],
                    z_hbm.at[pl.ds(base + CHUNK_PTS * (g - 2), CHUNK_PTS)],
                    sem_o).wait()

            def point_body(pp, c2):
                p = CHUNK_PTS * g + pp
                acc = [jnp.zeros((L,), jnp.float32) for _ in range(4 * 8)]
                for n in range(MAX_NB):
                    if n % 4 == 0:
                        wv = w_v[p, pl.ds(L * (n // 4), L)]
                    r = MAX_NB * pp + n
                    row = [rows_v[half, r, pl.ds(L * j, L)]
                           for j in range(8)]
                    for m in range(WN):
                        wb = _take(wv, _bcast_idx(4 * (n % 4) + m))
                        for j in range(8):
                            acc[8 * m + j] = acc[8 * m + j] + wb * row[j]
                for m in range(WN):
                    for j in range(8):
                        zstage_v[half, pp,
                                 pl.ds(IN_CH * m + L * j, L)] = acc[8 * m + j]
                return c2

            lax.fori_loop(0, CHUNK_PTS, point_body, 0)
            pltpu.async_copy(
                zstage_v.at[half],
                z_hbm.at[pl.ds(base + CHUNK_PTS * g, CHUNK_PTS)], sem_o)
            _issue_gather(g + 2, half)
        return carry

    lax.fori_loop(0, NCHUNK // 2, pair_body, 0)

    for half in range(2):
        _wait_gather(NCHUNK + half, half)
        pltpu.make_async_copy(
            zstage_v.at[half],
            z_hbm.at[pl.ds(base + CHUNK_PTS * (NCHUNK - 2 + half),
                           CHUNK_PTS)], sem_o).wait()


@jax.jit
def _sc_fuse(x, ids_pad, w_pad):
    mesh = plsc.VectorSubcoreMesh(core_axis_name="c", subcore_axis_name="s")
    return pl.kernel(
        _sc_fuse_body,
        out_type=jax.ShapeDtypeStruct((P_PAD, ZDIM), jnp.float32),
        mesh=mesh,
        scratch_types=[
            pltpu.VMEM((PTS_PER_W, IN_CH), jnp.float32),
            pltpu.VMEM((NCHUNK + 2, CHUNK_PTS, MAX_NB), jnp.int32),
            pltpu.VMEM((2, EDGES_PER_CHUNK, IN_CH), jnp.float32),
            pltpu.VMEM((2, CHUNK_PTS, ZDIM), jnp.float32),
            pltpu.SemaphoreType.DMA,
            pltpu.SemaphoreType.DMA,
            pltpu.SemaphoreType.DMA,
        ],
    )(x, ids_pad, w_pad)


def _tc_body(z_ref, w_ref, b_ref, o_ref):
    y = jnp.dot(z_ref[...], w_ref[...],
                preferred_element_type=jnp.float32) + b_ref[...]
    o_ref[...] = jnp.where(y > 0.0, y, jnp.exp(jnp.minimum(y, 0.0)) - 1.0)


@jax.jit
def _tc_matmul(z, wbig, bias):
    blk = 512
    grid = P_PAD // blk
    return pl.pallas_call(
        _tc_body,
        grid=(grid,),
        in_specs=[
            pl.BlockSpec((blk, ZDIM), lambda i: (i, 0)),
            pl.BlockSpec((ZDIM, OUT_CH), lambda i: (0, 0)),
            pl.BlockSpec((blk, OUT_CH), lambda i: (i, 0)),
        ],
        out_specs=pl.BlockSpec((blk, OUT_CH), lambda i: (i, 0)),
        out_shape=jax.ShapeDtypeStruct((OUT_PN, OUT_CH), jnp.float32),
    )(z, wbig, bias)


def kernel(in_pc, raw_w_weights, weights, bias, neighbor_id_lstlst):
    x = in_pc.reshape(IN_PN, IN_CH)
    ids_pad = jnp.pad(neighbor_id_lstlst, ((0, P_PAD - OUT_PN), (0, 0)),
                      constant_values=PAD_ID).reshape(
                          P_PAD // CHUNK_PTS, CHUNK_PTS, MAX_NB)
    w_pad = jnp.pad(raw_w_weights.reshape(OUT_PN, MAX_NB * WN),
                    ((0, P_PAD - OUT_PN), (0, 0)))
    wbig = weights.reshape(WN, OUT_CH, IN_CH).transpose(0, 2, 1).reshape(
        ZDIM, OUT_CH)
    z = _sc_fuse(x, ids_pad, w_pad)
    out = _tc_matmul(z, wbig, bias)
    return out.reshape(BATCH, OUT_PN, OUT_CH)
